# Initial kernel scaffold; baseline (speedup 1.0000x reference)
import jax
import jax.numpy as jnp
from jax.experimental import pallas as pl


def kernel(x, edge_index, W1_l, b1_l, W1_r, W2_l, b2_l, W2_r):
    raise NotImplementedError("write your pallas kernel here")


# speedup vs baseline: 11.8471x; 11.8471x over previous
import functools

import jax
import jax.numpy as jnp
from jax import lax
from jax.experimental import pallas as pl
from jax.experimental.pallas import tpu as pltpu
from jax.experimental.pallas import tpu_sc as plsc

N = 10000
E = 320000
D = 128

NC = 2
NS = 16
NW = NC * NS
E_PER_W = E // NW
K = 80
NCH = E_PER_W // K
NG = 5
G = NCH // NG
N_PAD = 10240
ROWS_PER_TILE = N_PAD // NS


def _sc_segment_sum(x, src, dst, zrows):
    f32 = jnp.float32

    @functools.partial(
        pl.kernel,
        out_type=jax.ShapeDtypeStruct((NC, N_PAD, D), f32),
        mesh=plsc.VectorSubcoreMesh(core_axis_name="c", subcore_axis_name="s"),
        scratch_types=(
            pltpu.VMEM((G, K), jnp.int32),
            pltpu.VMEM((G, K), jnp.int32),
            pltpu.VMEM((K, D), f32),
            pltpu.VMEM((K, D), f32),
            pltpu.VMEM_SHARED((N_PAD, D), f32),
            pltpu.SemaphoreType.DMA,
            pltpu.SemaphoreType.DMA,
        ),
        compiler_params=pltpu.CompilerParams(needs_layout_passes=False),
    )
    def body(x_hbm, src_hbm, dst_hbm, z_hbm, agg_hbm,
             src_v, dst_v, m0, m1, agg_sh, sem0, sem1):
        cid = lax.axis_index("c")
        sid = lax.axis_index("s")
        wid = cid * NS + sid
        row0 = sid * ROWS_PER_TILE

        pltpu.sync_copy(z_hbm, agg_sh.at[pl.ds(row0, ROWS_PER_TILE)])
        plsc.subcore_barrier()

        def gather_start(c, mbuf, sem):
            pltpu.async_copy(x_hbm.at[src_v.at[c]], mbuf, sem)

        def gather_wait(c, mbuf, sem):
            pltpu.make_async_copy(x_hbm.at[src_v.at[c]], mbuf, sem).wait()

        def scatter(c, mbuf):
            pltpu.sync_copy(mbuf, agg_sh.at[dst_v.at[c]], add=True)

        @pl.loop(0, NG)
        def _(g):
            pltpu.sync_copy(src_hbm.at[wid, g], src_v)
            pltpu.sync_copy(dst_hbm.at[wid, g], dst_v)
            gather_start(0, m0, sem0)

            @pl.loop(0, G - 1, step=2)
            def _(c):
                gather_start(c + 1, m1, sem1)
                gather_wait(c, m0, sem0)
                scatter(c, m0)
                gather_start(c + 2, m0, sem0)
                gather_wait(c + 1, m1, sem1)
                scatter(c + 1, m1)

            gather_wait(G - 1, m0, sem0)
            scatter(G - 1, m0)

        plsc.subcore_barrier()
        pltpu.sync_copy(agg_sh.at[pl.ds(row0, ROWS_PER_TILE)],
                        agg_hbm.at[cid, pl.ds(row0, ROWS_PER_TILE)])

    return body(x, src, dst, zrows)


def _sc_degree(dst_flat):
    f32 = jnp.float32

    @functools.partial(
        pl.kernel,
        out_type=jax.ShapeDtypeStruct((NW * N,), f32),
        mesh=plsc.VectorSubcoreMesh(core_axis_name="c", subcore_axis_name="s"),
        scratch_types=(
            pltpu.VMEM((E_PER_W,), jnp.int32),
            pltpu.VMEM((N,), f32),
        ),
        compiler_params=pltpu.CompilerParams(needs_layout_passes=False),
    )
    def body(dst_hbm, deg_hbm, dst_v, deg_v):
        cid = lax.axis_index("c")
        sid = lax.axis_index("s")
        wid = cid * NS + sid
        pltpu.sync_copy(dst_hbm.at[pl.ds(wid * E_PER_W, E_PER_W)], dst_v)
        zero16 = jnp.zeros((16,), f32)

        @pl.loop(0, N // 16)
        def _(i):
            deg_v[pl.ds(i * 16, 16)] = zero16

        ones16 = jnp.ones((16,), f32)

        @pl.loop(0, E_PER_W // 16)
        def _(j):
            dv = dst_v[pl.ds(j * 16, 16)]
            plsc.addupdate_scatter(deg_v, (dv,), ones16)

        pltpu.sync_copy(deg_v, deg_hbm.at[pl.ds(wid * N, N)])

    return body(dst_flat)


BLK = 2000


def _tc_rdeg(deg_p):
    def body(deg_ref, rd_ref):
        deg = jnp.sum(deg_ref[...], axis=0)
        rd_ref[...] = (1.0 / jnp.maximum(deg, 1.0))[:, None]

    return pl.pallas_call(
        body,
        out_shape=jax.ShapeDtypeStruct((N, 1), jnp.float32),
    )(deg_p)


def _tc_combine(agg_p, rdeg, x, w_l, b_l, w_r):
    def body(agg_ref, rd_ref, x_ref, wl_ref, bl_ref, wr_ref, h_ref):
        a = (agg_ref[0] + agg_ref[1]) * rd_ref[...]
        h = (lax.dot_general(a, wl_ref[...], (((1,), (1,)), ((), ())),
                             preferred_element_type=jnp.float32)
             + bl_ref[...][None, :]
             + lax.dot_general(x_ref[...], wr_ref[...], (((1,), (1,)), ((), ())),
                               preferred_element_type=jnp.float32))
        h_ref[...] = jnp.maximum(h, 0.0)

    return pl.pallas_call(
        body,
        grid=(N // BLK,),
        in_specs=[
            pl.BlockSpec((NC, BLK, D), lambda i: (0, i, 0)),
            pl.BlockSpec((BLK, 1), lambda i: (i, 0)),
            pl.BlockSpec((BLK, D), lambda i: (i, 0)),
            pl.BlockSpec((D, D), lambda i: (0, 0)),
            pl.BlockSpec((D,), lambda i: (0,)),
            pl.BlockSpec((D, D), lambda i: (0, 0)),
        ],
        out_specs=pl.BlockSpec((BLK, D), lambda i: (i, 0)),
        out_shape=jax.ShapeDtypeStruct((N, D), jnp.float32),
    )(agg_p, rdeg, x, w_l, b_l, w_r)


def kernel(x, edge_index, W1_l, b1_l, W1_r, W2_l, b2_l, W2_r):
    ei = edge_index.astype(jnp.int32)
    src = ei[0].reshape(NW, NG, G, K)
    dst = ei[1].reshape(NW, NG, G, K)
    zrows = jnp.zeros((ROWS_PER_TILE, D), jnp.float32)

    deg = _sc_degree(ei[1])
    rdeg = _tc_rdeg(deg.reshape(NW, N))
    agg1 = _sc_segment_sum(x, src, dst, zrows)
    h = _tc_combine(agg1, rdeg, x, W1_l, b1_l, W1_r)
    agg2 = _sc_segment_sum(h, src, dst, zrows)
    out = _tc_combine(agg2, rdeg, h, W2_l, b2_l, W2_r)
    return out
